# baseline (speedup 1.0000x reference)
import functools

import jax
import jax.numpy as jnp
from jax.experimental import pallas as pl
from jax.experimental.pallas import tpu as pltpu


def _vmem_gather_kernel(lbl_ref, drop_ref, table_ref, out_ref,
                        *, tile_b: int, cfg_row: int):
    base = pl.program_id(1) * tile_b
    for r in range(tile_b):
        lbl = lbl_ref[base + r]
        drop = drop_ref[base + r]
        row = jnp.where(drop == 1, cfg_row, lbl)
        row = jnp.clip(row, 0, cfg_row)
        base8 = pl.multiple_of((row >> 3) << 3, 8)
        sub = row & 7
        chunk = table_ref[pl.ds(base8, 8), :]
        rot = pltpu.roll(chunk, (8 - sub) & 7, axis=0)
        out_ref[pl.ds(r, 1), :] = rot[0:1, :]


def kernel(labels, table, force_drop_ids):
    (B,) = labels.shape
    V, H = table.shape
    cfg_row = V - 1

    labels = labels.astype(jnp.int32)
    force_drop_ids = force_drop_ids.astype(jnp.int32)

    tile_b = 256
    while B % tile_b != 0:
        tile_b //= 2
    n_b = B // tile_b

    tile_h = H
    for cand in (3, 2, 4, 1):
        if H % (cand * 128) == 0 and (H // cand) % 128 == 0:
            tile_h = H // cand
            break
    n_h = H // tile_h
    itemsize = jnp.dtype(table.dtype).itemsize

    grid_spec = pltpu.PrefetchScalarGridSpec(
        num_scalar_prefetch=2,
        grid=(n_h, n_b),
        in_specs=[
            pl.BlockSpec((V, tile_h), lambda h, b, lbl, drp: (0, h)),
        ],
        out_specs=pl.BlockSpec((tile_b, tile_h),
                               lambda h, b, lbl, drp: (b, h)),
    )
    out = pl.pallas_call(
        functools.partial(_vmem_gather_kernel, tile_b=tile_b, cfg_row=cfg_row),
        out_shape=jax.ShapeDtypeStruct((B, H), table.dtype),
        grid_spec=grid_spec,
        compiler_params=pltpu.CompilerParams(
            dimension_semantics=("parallel", "arbitrary"),
            vmem_limit_bytes=100 * 1024 * 1024,
            disable_bounds_checks=True,
        ),
        cost_estimate=pl.CostEstimate(
            flops=0,
            transcendentals=0,
            bytes_accessed=(V * H + B * H) * itemsize + 8 * B),
    )(labels, force_drop_ids, table)
    return out
